# Initial kernel scaffold; baseline (speedup 1.0000x reference)
import jax
import jax.numpy as jnp
from jax.experimental import pallas as pl


def kernel(hidden_states, W1, b1, W2, b2, Wt, bt, type_values):
    raise NotImplementedError("write your pallas kernel here")


# speedup vs baseline: 2.0438x; 2.0438x over previous
import functools

import jax
import jax.numpy as jnp
from jax import lax
from jax.experimental import pallas as pl
from jax.experimental.pallas import tpu as pltpu

K = 512
SCALE = 0.1
BS = 512
CJ = 512


def _score_body(b1_ref, b2_ref, bt_ref, x_ref, w1p_ref, w2p_ref, wtp_ref,
                sc_ref, ty_ref):
    x = x_ref[0]
    acc = jnp.dot(x, w1p_ref[...], preferred_element_type=jnp.float32)
    h = jnp.maximum(acc[:, :64] + b1_ref[...], 0.0)
    sc = jnp.dot(h, w2p_ref[...], preferred_element_type=jnp.float32)
    sc_ref[...] = sc[:, 0:1] + b2_ref[0]
    lg = jnp.dot(x, wtp_ref[...], preferred_element_type=jnp.float32)
    l0 = lg[:, 0:1] + bt_ref[0]
    l1 = lg[:, 1:2] + bt_ref[1]
    l2 = lg[:, 2:3] + bt_ref[2]
    t0 = (l0 >= l1) & (l0 >= l2)
    t1 = jnp.logical_not(t0) & (l1 >= l2)
    ty_ref[...] = jnp.where(t0, 0, jnp.where(t1, 1, 2)).astype(jnp.int32)


def _topk_body(tv_ref, srow_ref, scol_ref, tycol_ref,
               idx_ref, typ_ref, msk_ref, val_ref, *, seq):
    srow = srow_ref[0]
    scol = scol_ref[0]
    icol = lax.broadcasted_iota(jnp.int32, (seq, 1), 0)
    ranks = jnp.zeros((seq, 1), jnp.int32)
    for c in range(seq // CJ):
        sj = srow[:, c * CJ:(c + 1) * CJ]
        jj = lax.broadcasted_iota(jnp.int32, (1, CJ), 1) + c * CJ
        beats = (sj > scol) | ((sj == scol) & (jj < icol))
        ranks = ranks + jnp.sum(beats.astype(jnp.int32), axis=1,
                                keepdims=True)
    kk = lax.broadcasted_iota(jnp.int32, (1, K), 1)
    sel = ranks == kk
    tyc = tycol_ref[0]
    idx_ref[0] = jnp.sum(jnp.where(sel, icol, 0), axis=0, keepdims=True)
    typ_ref[0] = jnp.sum(jnp.where(sel, tyc, 0), axis=0, keepdims=True)
    msk_ref[0] = (ranks < K).astype(jnp.int32)
    vcol = jnp.where(tyc == 0, tv_ref[0],
                     jnp.where(tyc == 1, tv_ref[1], tv_ref[2])) * SCALE
    val_ref[0] = vcol


def _apply_body(x_ref, v_ref, m_ref, o_ref):
    xv = x_ref[0]
    vc = v_ref[...]
    keep = m_ref[...] == 0
    o_ref[0] = jnp.where(keep, xv, jnp.broadcast_to(vc, xv.shape))


def kernel(hidden_states, W1, b1, W2, b2, Wt, bt, type_values):
    x = hidden_states.astype(jnp.float32)
    bsz, seq, hid = x.shape
    nbs = seq // BS
    sel_h = W1.shape[1]

    w1p = jnp.zeros((hid, 128), jnp.float32).at[:, :sel_h].set(W1)
    w2p = jnp.zeros((sel_h, 128), jnp.float32).at[:, 0:1].set(W2)
    wtp = jnp.zeros((hid, 128), jnp.float32).at[:, 0:3].set(Wt)
    b1r = b1.reshape(1, sel_h)

    scores, types = pl.pallas_call(
        _score_body,
        grid=(bsz, nbs),
        in_specs=[
            pl.BlockSpec((1, sel_h), lambda b, n: (0, 0)),
            pl.BlockSpec(memory_space=pltpu.SMEM),
            pl.BlockSpec(memory_space=pltpu.SMEM),
            pl.BlockSpec((1, BS, hid), lambda b, n: (b, n, 0)),
            pl.BlockSpec((hid, 128), lambda b, n: (0, 0)),
            pl.BlockSpec((sel_h, 128), lambda b, n: (0, 0)),
            pl.BlockSpec((hid, 128), lambda b, n: (0, 0)),
        ],
        out_specs=[
            pl.BlockSpec((BS, 1), lambda b, n: (b * nbs + n, 0)),
            pl.BlockSpec((BS, 1), lambda b, n: (b * nbs + n, 0)),
        ],
        out_shape=[
            jax.ShapeDtypeStruct((bsz * seq, 1), jnp.float32),
            jax.ShapeDtypeStruct((bsz * seq, 1), jnp.int32),
        ],
    )(b1r, b2, bt, x, w1p, w2p, wtp)

    sel_idx, sel_typ, msk, vals = pl.pallas_call(
        functools.partial(_topk_body, seq=seq),
        grid=(bsz,),
        in_specs=[
            pl.BlockSpec(memory_space=pltpu.SMEM),
            pl.BlockSpec((1, 1, seq), lambda b: (b, 0, 0)),
            pl.BlockSpec((1, seq, 1), lambda b: (b, 0, 0)),
            pl.BlockSpec((1, seq, 1), lambda b: (b, 0, 0)),
        ],
        out_specs=[
            pl.BlockSpec((1, 1, K), lambda b: (b, 0, 0)),
            pl.BlockSpec((1, 1, K), lambda b: (b, 0, 0)),
            pl.BlockSpec((1, seq, 1), lambda b: (b, 0, 0)),
            pl.BlockSpec((1, seq, 1), lambda b: (b, 0, 0)),
        ],
        out_shape=[
            jax.ShapeDtypeStruct((bsz, 1, K), jnp.int32),
            jax.ShapeDtypeStruct((bsz, 1, K), jnp.int32),
            jax.ShapeDtypeStruct((bsz, seq, 1), jnp.int32),
            jax.ShapeDtypeStruct((bsz, seq, 1), jnp.float32),
        ],
    )(type_values, scores.reshape(bsz, 1, seq), scores.reshape(bsz, seq, 1),
      types.reshape(bsz, seq, 1))

    xi16 = lax.bitcast_convert_type(x.astype(jnp.float16), jnp.int16)
    vi16 = lax.bitcast_convert_type(
        vals.reshape(bsz * seq, 1).astype(jnp.float16), jnp.int16)

    out_i16 = pl.pallas_call(
        _apply_body,
        grid=(bsz, nbs),
        in_specs=[
            pl.BlockSpec((1, BS, hid), lambda b, n: (b, n, 0)),
            pl.BlockSpec((BS, 1), lambda b, n: (b * nbs + n, 0)),
            pl.BlockSpec((BS, 1), lambda b, n: (b * nbs + n, 0)),
        ],
        out_specs=pl.BlockSpec((1, BS, hid), lambda b, n: (b, n, 0)),
        out_shape=jax.ShapeDtypeStruct((bsz, seq, hid), jnp.int16),
        compiler_params=pltpu.CompilerParams(
            allow_input_fusion=[True, False, False]),
    )(xi16, vi16, msk.reshape(bsz * seq, 1))
    perturbed = lax.bitcast_convert_type(out_i16, jnp.float16)

    return (perturbed, sel_idx.reshape(bsz, K), sel_typ.reshape(bsz, K),
            jnp.zeros((bsz,), jnp.float32))
